# baseline (speedup 1.0000x reference)
import functools

import jax
import jax.numpy as jnp
from jax import lax
from jax.experimental import pallas as pl
from jax.experimental.pallas import tpu as pltpu
from jax.experimental.pallas import tpu_sc as plsc

BATCH = 16384
NF = 64
NC, NS, LANES = 2, 16, 16
NW = NC * NS
BPW = BATCH // NW
CHUNK = 128
NCH = BPW // CHUNK


def _nmf_body(iw_hbm, io_hbm, w_hbm, h_hbm, wb_hbm, hb_hbm, out_hbm,
              iw_v, io_v, rw_v, rh_v, bw_v, bh_v, o_v, sem):
  wid = lax.axis_index("s") * NC + lax.axis_index("c")

  pltpu.sync_copy(iw_hbm.at[wid], iw_v)
  pltpu.sync_copy(io_hbm.at[wid], io_v)

  copies = []
  for j in range(NCH):
    sl = pl.ds(j * CHUNK, CHUNK)
    copies.append(pltpu.async_copy(w_hbm.at[iw_v.at[j]], rw_v.at[sl], sem))
    copies.append(pltpu.async_copy(h_hbm.at[io_v.at[j]], rh_v.at[sl], sem))
    copies.append(pltpu.async_copy(wb_hbm.at[iw_v.at[j]], bw_v.at[sl], sem))
    copies.append(pltpu.async_copy(hb_hbm.at[io_v.at[j]], bh_v.at[sl], sem))
  for c in copies:
    c.wait()

  lanes = lax.iota(jnp.int32, LANES)

  def group(g, carry):
    base = g * LANES
    acc = jnp.zeros((LANES,), jnp.float32)
    for i in range(LANES):
      p = base + i
      s = rw_v[p, pl.ds(0, LANES)] * rh_v[p, pl.ds(0, LANES)]
      for k in range(1, NF // LANES):
        s = s + rw_v[p, pl.ds(k * LANES, LANES)] * rh_v[p, pl.ds(k * LANES, LANES)]
      acc = jnp.where(lanes == i, jnp.sum(s), acc)
    o_v[pl.ds(base, LANES)] = acc + bw_v[pl.ds(base, LANES)] + bh_v[pl.ds(base, LANES)]
    return carry

  lax.fori_loop(0, BPW // LANES, group, 0)

  pltpu.sync_copy(o_v, out_hbm.at[pl.ds(wid * BPW, BPW)])


def kernel(nodes, W, H, w_bias, h_bias):
  nodes = nodes.astype(jnp.int32)
  iw = nodes[:, 0].reshape(NW, NCH, CHUNK)
  io = nodes[:, 1].reshape(NW, NCH, CHUNK)

  mesh = plsc.VectorSubcoreMesh(core_axis_name="c", subcore_axis_name="s",
                                num_cores=NC, num_subcores=NS)
  f = pl.kernel(
      _nmf_body,
      out_type=jax.ShapeDtypeStruct((BATCH,), jnp.float32),
      mesh=mesh,
      compiler_params=pltpu.CompilerParams(needs_layout_passes=False,
                                           use_tc_tiling_on_sc=False),
      scratch_types=[
          pltpu.VMEM((NCH, CHUNK), jnp.int32),
          pltpu.VMEM((NCH, CHUNK), jnp.int32),
          pltpu.VMEM((BPW, NF), jnp.float32),
          pltpu.VMEM((BPW, NF), jnp.float32),
          pltpu.VMEM((BPW,), jnp.float32),
          pltpu.VMEM((BPW,), jnp.float32),
          pltpu.VMEM((BPW,), jnp.float32),
          pltpu.SemaphoreType.DMA,
      ],
  )
  return f(iw, io, W, H, w_bias.reshape(-1), h_bias.reshape(-1))
